# baseline (speedup 1.0000x reference)
import functools
import math

import jax
import jax.numpy as jnp
from jax import lax
from jax.experimental import pallas as pl
from jax.experimental.pallas import tpu as pltpu
from jax.experimental.pallas import tpu_sc as plsc

_EMB = 128
_SCALE = math.sqrt(float(_EMB))

_NC = 2
_NS = 16
_NW = _NC * _NS


@functools.lru_cache(maxsize=None)
def _sc_gather(n_rows: int):
    b_per_w = n_rows // _NW
    mesh = plsc.VectorSubcoreMesh(core_axis_name="c", subcore_axis_name="s")

    @functools.partial(
        pl.kernel,
        out_type=jax.ShapeDtypeStruct((n_rows, _EMB), jnp.float32),
        mesh=mesh,
        scratch_types=[
            pltpu.VMEM((b_per_w,), jnp.int32),
            pltpu.VMEM((b_per_w, _EMB), jnp.float32),
            pltpu.SemaphoreType.DMA,
        ],
    )
    def gather(table_hbm, idx_hbm, out_hbm, idx_v, rows_v, sem):
        wid = lax.axis_index("s") * _NC + lax.axis_index("c")
        base = wid * b_per_w
        pltpu.sync_copy(idx_hbm.at[pl.ds(base, b_per_w)], idx_v)
        pltpu.async_copy(table_hbm.at[idx_v], rows_v, sem).wait()
        pltpu.sync_copy(rows_v, out_hbm.at[pl.ds(base, b_per_w)])

    return gather


def _add_body(x_ref, enc_ref, o_ref):
    o_ref[...] = x_ref[...] + enc_ref[...] * _SCALE


_ROWS_PER_BLOCK = 32

_CHUNK_ROWS = 8
_NBUF = 6


@functools.lru_cache(maxsize=None)
def _tc_add(n_rows: int, tokens: int):
    r = _ROWS_PER_BLOCK
    return pl.pallas_call(
        _add_body,
        grid=(n_rows // r,),
        in_specs=[
            pl.BlockSpec((r, tokens, _EMB), lambda i: (i, 0, 0)),
            pl.BlockSpec((r, 1, _EMB), lambda i: (i, 0, 0)),
        ],
        out_specs=pl.BlockSpec((r, tokens, _EMB), lambda i: (i, 0, 0)),
        out_shape=jax.ShapeDtypeStruct((n_rows, tokens, _EMB), jnp.float32),
    )


@functools.lru_cache(maxsize=None)
def _tc_add_manual(n_rows: int, tokens: int, interpret: bool = False):
    cr = _CHUNK_ROWS
    nb = _NBUF
    k_ahead = nb - 1
    n_chunks = n_rows // cr

    def body(x_hbm, enc_vmem, o_hbm, in_buf, out_buf, sem_in, sem_out):
        i = pl.program_id(0)
        sl = jax.lax.rem(i, nb)

        def read(chunk, slot):
            return pltpu.make_async_copy(
                x_hbm.at[pl.ds(chunk * cr, cr)], in_buf.at[slot],
                sem_in.at[slot])

        def write(chunk, slot):
            return pltpu.make_async_copy(
                out_buf.at[slot], o_hbm.at[pl.ds(chunk * cr, cr)],
                sem_out.at[slot])

        @pl.when(i == 0)
        def _():
            for k in range(k_ahead):
                read(k, k).start()

        @pl.when(i + k_ahead < n_chunks)
        def _():
            slot = jax.lax.rem(i + k_ahead, nb)
            read(i + k_ahead, slot).start()

        read(i, sl).wait()

        @pl.when(i >= nb)
        def _():
            write(i - nb, sl).wait()

        out_buf[sl] = in_buf[sl] + enc_vmem[pl.ds(i * cr, cr)] * _SCALE
        write(i, sl).start()

        @pl.when(i == n_chunks - 1)
        def _():
            for k in range(nb):
                chunk = n_chunks - nb + k
                write(chunk, jax.lax.rem(jnp.int32(chunk), nb)).wait()

    return pl.pallas_call(
        body,
        grid=(n_chunks,),
        in_specs=[
            pl.BlockSpec(memory_space=pl.ANY),
            pl.BlockSpec(memory_space=pltpu.VMEM),
        ],
        out_specs=pl.BlockSpec(memory_space=pl.ANY),
        out_shape=jax.ShapeDtypeStruct((n_rows, tokens, _EMB), jnp.float32),
        scratch_shapes=[
            pltpu.VMEM((nb, cr, tokens, _EMB), jnp.float32),
            pltpu.VMEM((nb, cr, tokens, _EMB), jnp.float32),
            pltpu.SemaphoreType.DMA((nb,)),
            pltpu.SemaphoreType.DMA((nb,)),
        ],
        interpret=interpret,
    )


def kernel(x, blocks, emb_table):
    batch, num_blocks, tokens, emb = x.shape
    n_rows = batch * num_blocks
    idx = blocks.astype(jnp.int32).reshape(n_rows)
    enc = _sc_gather(n_rows)(emb_table, idx)
    out = _tc_add_manual(n_rows, tokens)(
        x.reshape(n_rows, tokens, emb), enc.reshape(n_rows, 1, emb)
    )
    return out.reshape(batch, num_blocks * tokens, emb)


# speedup vs baseline: 1.0030x; 1.0030x over previous
import functools
import math

import jax
import jax.numpy as jnp
from jax import lax
from jax.experimental import pallas as pl
from jax.experimental.pallas import tpu as pltpu
from jax.experimental.pallas import tpu_sc as plsc

_EMB = 128
_SCALE = math.sqrt(float(_EMB))

_NC = 2
_NS = 16
_NW = _NC * _NS


@functools.lru_cache(maxsize=None)
def _sc_gather(n_rows: int):
    b_per_w = n_rows // _NW
    mesh = plsc.VectorSubcoreMesh(core_axis_name="c", subcore_axis_name="s")

    @functools.partial(
        pl.kernel,
        out_type=jax.ShapeDtypeStruct((n_rows, _EMB), jnp.float32),
        mesh=mesh,
        scratch_types=[
            pltpu.VMEM((b_per_w,), jnp.int32),
            pltpu.VMEM((b_per_w, _EMB), jnp.float32),
            pltpu.SemaphoreType.DMA,
        ],
        compiler_params=pltpu.CompilerParams(use_tc_tiling_on_sc=True),
    )
    def gather(table_hbm, idx_hbm, out_hbm, idx_v, rows_v, sem):
        wid = lax.axis_index("s") * _NC + lax.axis_index("c")
        base = wid * b_per_w
        pltpu.sync_copy(idx_hbm.at[pl.ds(base, b_per_w)], idx_v)
        pltpu.async_copy(table_hbm.at[idx_v], rows_v, sem).wait()
        pltpu.sync_copy(rows_v, out_hbm.at[pl.ds(base, b_per_w)])

    return gather


def _add_body(x_ref, enc_ref, o_ref):
    o_ref[...] = x_ref[...] + enc_ref[...] * _SCALE


_ROWS_PER_BLOCK = 32

_CHUNK_ROWS = 8
_NBUF = 6


@functools.lru_cache(maxsize=None)
def _tc_add(n_rows: int, tokens: int):
    r = _ROWS_PER_BLOCK
    return pl.pallas_call(
        _add_body,
        grid=(n_rows // r,),
        in_specs=[
            pl.BlockSpec((r, tokens, _EMB), lambda i: (i, 0, 0)),
            pl.BlockSpec((r, 1, _EMB), lambda i: (i, 0, 0)),
        ],
        out_specs=pl.BlockSpec((r, tokens, _EMB), lambda i: (i, 0, 0)),
        out_shape=jax.ShapeDtypeStruct((n_rows, tokens, _EMB), jnp.float32),
    )


@functools.lru_cache(maxsize=None)
def _tc_add_manual(n_rows: int, tokens: int, interpret: bool = False):
    cr = _CHUNK_ROWS
    nb = _NBUF
    k_ahead = nb - 1
    n_chunks = n_rows // cr

    def body(x_hbm, enc_vmem, o_hbm, in_buf, out_buf, sem_in, sem_out):
        i = pl.program_id(0)
        sl = jax.lax.rem(i, nb)

        def read(chunk, slot):
            return pltpu.make_async_copy(
                x_hbm.at[pl.ds(chunk * cr, cr)], in_buf.at[slot],
                sem_in.at[slot])

        def write(chunk, slot):
            return pltpu.make_async_copy(
                out_buf.at[slot], o_hbm.at[pl.ds(chunk * cr, cr)],
                sem_out.at[slot])

        @pl.when(i == 0)
        def _():
            for k in range(k_ahead):
                read(k, k).start()

        @pl.when(i + k_ahead < n_chunks)
        def _():
            slot = jax.lax.rem(i + k_ahead, nb)
            read(i + k_ahead, slot).start()

        read(i, sl).wait()

        @pl.when(i >= nb)
        def _():
            write(i - nb, sl).wait()

        out_buf[sl] = in_buf[sl] + enc_vmem[pl.ds(i * cr, cr)] * _SCALE
        write(i, sl).start()

        @pl.when(i == n_chunks - 1)
        def _():
            for k in range(nb):
                chunk = n_chunks - nb + k
                write(chunk, jax.lax.rem(jnp.int32(chunk), nb)).wait()

    return pl.pallas_call(
        body,
        grid=(n_chunks,),
        in_specs=[
            pl.BlockSpec(memory_space=pl.ANY),
            pl.BlockSpec(memory_space=pltpu.VMEM),
        ],
        out_specs=pl.BlockSpec(memory_space=pl.ANY),
        out_shape=jax.ShapeDtypeStruct((n_rows, tokens, _EMB), jnp.float32),
        scratch_shapes=[
            pltpu.VMEM((nb, cr, tokens, _EMB), jnp.float32),
            pltpu.VMEM((nb, cr, tokens, _EMB), jnp.float32),
            pltpu.SemaphoreType.DMA((nb,)),
            pltpu.SemaphoreType.DMA((nb,)),
        ],
        interpret=interpret,
    )


def kernel(x, blocks, emb_table):
    batch, num_blocks, tokens, emb = x.shape
    n_rows = batch * num_blocks
    idx = blocks.astype(jnp.int32).reshape(n_rows)
    enc = _sc_gather(n_rows)(emb_table, idx)
    out = _tc_add(n_rows, tokens)(
        x.reshape(n_rows, tokens, emb), enc.reshape(n_rows, 1, emb)
    )
    return out.reshape(batch, num_blocks * tokens, emb)


# speedup vs baseline: 1.0074x; 1.0043x over previous
import functools
import math

import jax
import jax.numpy as jnp
from jax import lax
from jax.experimental import pallas as pl
from jax.experimental.pallas import tpu as pltpu
from jax.experimental.pallas import tpu_sc as plsc

_EMB = 128
_SCALE = math.sqrt(float(_EMB))

_NC = 2
_NS = 16
_NW = _NC * _NS


@functools.lru_cache(maxsize=None)
def _sc_gather(n_rows: int):
    n_cores = 1
    b_per_w = n_rows // (n_cores * _NS)
    mesh = plsc.VectorSubcoreMesh(
        core_axis_name="c", subcore_axis_name="s", num_cores=n_cores)

    @functools.partial(
        pl.kernel,
        out_type=jax.ShapeDtypeStruct((n_rows, _EMB), jnp.float32),
        mesh=mesh,
        scratch_types=[
            pltpu.VMEM((b_per_w,), jnp.int32),
            pltpu.VMEM((b_per_w, _EMB), jnp.float32),
            pltpu.SemaphoreType.DMA,
        ],
        compiler_params=pltpu.CompilerParams(use_tc_tiling_on_sc=True),
    )
    def gather(table_hbm, idx_hbm, out_hbm, idx_v, rows_v, sem):
        wid = lax.axis_index("s") * n_cores + lax.axis_index("c")
        base = wid * b_per_w
        pltpu.sync_copy(idx_hbm.at[pl.ds(base, b_per_w)], idx_v)
        pltpu.async_copy(table_hbm.at[idx_v], rows_v, sem).wait()
        pltpu.sync_copy(rows_v, out_hbm.at[pl.ds(base, b_per_w)])

    return gather


def _add_body(x_ref, enc_ref, o_ref):
    o_ref[...] = x_ref[...] + enc_ref[...] * _SCALE


_ROWS_PER_BLOCK = 32

_CHUNK_ROWS = 8
_NBUF = 6


@functools.lru_cache(maxsize=None)
def _tc_add(n_rows: int, tokens: int):
    r = _ROWS_PER_BLOCK
    return pl.pallas_call(
        _add_body,
        grid=(n_rows // r,),
        in_specs=[
            pl.BlockSpec((r, tokens, _EMB), lambda i: (i, 0, 0)),
            pl.BlockSpec((r, 1, _EMB), lambda i: (i, 0, 0)),
        ],
        out_specs=pl.BlockSpec((r, tokens, _EMB), lambda i: (i, 0, 0)),
        out_shape=jax.ShapeDtypeStruct((n_rows, tokens, _EMB), jnp.float32),
        compiler_params=pltpu.CompilerParams(vmem_limit_bytes=100 * 1024 * 1024),
    )


@functools.lru_cache(maxsize=None)
def _tc_add_manual(n_rows: int, tokens: int, interpret: bool = False):
    cr = _CHUNK_ROWS
    nb = _NBUF
    k_ahead = nb - 1
    n_chunks = n_rows // cr

    def body(x_hbm, enc_vmem, o_hbm, in_buf, out_buf, sem_in, sem_out):
        i = pl.program_id(0)
        sl = jax.lax.rem(i, nb)

        def read(chunk, slot):
            return pltpu.make_async_copy(
                x_hbm.at[pl.ds(chunk * cr, cr)], in_buf.at[slot],
                sem_in.at[slot])

        def write(chunk, slot):
            return pltpu.make_async_copy(
                out_buf.at[slot], o_hbm.at[pl.ds(chunk * cr, cr)],
                sem_out.at[slot])

        @pl.when(i == 0)
        def _():
            for k in range(k_ahead):
                read(k, k).start()

        @pl.when(i + k_ahead < n_chunks)
        def _():
            slot = jax.lax.rem(i + k_ahead, nb)
            read(i + k_ahead, slot).start()

        read(i, sl).wait()

        @pl.when(i >= nb)
        def _():
            write(i - nb, sl).wait()

        out_buf[sl] = in_buf[sl] + enc_vmem[pl.ds(i * cr, cr)] * _SCALE
        write(i, sl).start()

        @pl.when(i == n_chunks - 1)
        def _():
            for k in range(nb):
                chunk = n_chunks - nb + k
                write(chunk, jax.lax.rem(jnp.int32(chunk), nb)).wait()

    return pl.pallas_call(
        body,
        grid=(n_chunks,),
        in_specs=[
            pl.BlockSpec(memory_space=pl.ANY),
            pl.BlockSpec(memory_space=pltpu.VMEM),
        ],
        out_specs=pl.BlockSpec(memory_space=pl.ANY),
        out_shape=jax.ShapeDtypeStruct((n_rows, tokens, _EMB), jnp.float32),
        scratch_shapes=[
            pltpu.VMEM((nb, cr, tokens, _EMB), jnp.float32),
            pltpu.VMEM((nb, cr, tokens, _EMB), jnp.float32),
            pltpu.SemaphoreType.DMA((nb,)),
            pltpu.SemaphoreType.DMA((nb,)),
        ],
        interpret=interpret,
    )


def kernel(x, blocks, emb_table):
    batch, num_blocks, tokens, emb = x.shape
    n_rows = batch * num_blocks
    idx = blocks.astype(jnp.int32).reshape(n_rows)
    enc = _sc_gather(n_rows)(emb_table, idx)
    out = _tc_add(n_rows, tokens)(
        x.reshape(n_rows, tokens, emb), enc.reshape(n_rows, 1, emb)
    )
    return out.reshape(batch, num_blocks * tokens, emb)


# speedup vs baseline: 1.0081x; 1.0007x over previous
import functools
import math

import jax
import jax.numpy as jnp
from jax import lax
from jax.experimental import pallas as pl
from jax.experimental.pallas import tpu as pltpu
from jax.experimental.pallas import tpu_sc as plsc

_EMB = 128
_SCALE = math.sqrt(float(_EMB))

_NC = 2
_NS = 16
_NW = _NC * _NS


@functools.lru_cache(maxsize=None)
def _sc_gather(n_rows: int):
    n_cores = 1
    b_per_w = n_rows // (n_cores * _NS)
    mesh = plsc.VectorSubcoreMesh(
        core_axis_name="c", subcore_axis_name="s", num_cores=n_cores)

    @functools.partial(
        pl.kernel,
        out_type=jax.ShapeDtypeStruct((n_rows, _EMB), jnp.float32),
        mesh=mesh,
        scratch_types=[
            pltpu.VMEM((b_per_w,), jnp.int32),
            pltpu.VMEM((b_per_w, _EMB), jnp.float32),
            pltpu.SemaphoreType.DMA,
        ],
        compiler_params=pltpu.CompilerParams(
            use_tc_tiling_on_sc=True, skip_device_barrier=True),
    )
    def gather(table_hbm, idx_hbm, out_hbm, idx_v, rows_v, sem):
        wid = lax.axis_index("s") * n_cores + lax.axis_index("c")
        base = wid * b_per_w
        pltpu.sync_copy(idx_hbm.at[pl.ds(base, b_per_w)], idx_v)
        pltpu.async_copy(table_hbm.at[idx_v], rows_v, sem).wait()
        pltpu.sync_copy(rows_v, out_hbm.at[pl.ds(base, b_per_w)])

    return gather


def _add_body(x_ref, enc_ref, o_ref):
    o_ref[...] = x_ref[...] + enc_ref[...] * _SCALE


_ROWS_PER_BLOCK = 32

_CHUNK_ROWS = 8
_NBUF = 6


@functools.lru_cache(maxsize=None)
def _tc_add(n_rows: int, tokens: int):
    r = _ROWS_PER_BLOCK
    return pl.pallas_call(
        _add_body,
        grid=(n_rows // r,),
        in_specs=[
            pl.BlockSpec((r, tokens, _EMB), lambda i: (i, 0, 0)),
            pl.BlockSpec((r, 1, _EMB), lambda i: (i, 0, 0)),
        ],
        out_specs=pl.BlockSpec((r, tokens, _EMB), lambda i: (i, 0, 0)),
        out_shape=jax.ShapeDtypeStruct((n_rows, tokens, _EMB), jnp.float32),
        compiler_params=pltpu.CompilerParams(
            vmem_limit_bytes=100 * 1024 * 1024, skip_device_barrier=True),
    )


@functools.lru_cache(maxsize=None)
def _tc_add_manual(n_rows: int, tokens: int, interpret: bool = False):
    cr = _CHUNK_ROWS
    nb = _NBUF
    k_ahead = nb - 1
    n_chunks = n_rows // cr

    def body(x_hbm, enc_vmem, o_hbm, in_buf, out_buf, sem_in, sem_out):
        i = pl.program_id(0)
        sl = jax.lax.rem(i, nb)

        def read(chunk, slot):
            return pltpu.make_async_copy(
                x_hbm.at[pl.ds(chunk * cr, cr)], in_buf.at[slot],
                sem_in.at[slot])

        def write(chunk, slot):
            return pltpu.make_async_copy(
                out_buf.at[slot], o_hbm.at[pl.ds(chunk * cr, cr)],
                sem_out.at[slot])

        @pl.when(i == 0)
        def _():
            for k in range(k_ahead):
                read(k, k).start()

        @pl.when(i + k_ahead < n_chunks)
        def _():
            slot = jax.lax.rem(i + k_ahead, nb)
            read(i + k_ahead, slot).start()

        read(i, sl).wait()

        @pl.when(i >= nb)
        def _():
            write(i - nb, sl).wait()

        out_buf[sl] = in_buf[sl] + enc_vmem[pl.ds(i * cr, cr)] * _SCALE
        write(i, sl).start()

        @pl.when(i == n_chunks - 1)
        def _():
            for k in range(nb):
                chunk = n_chunks - nb + k
                write(chunk, jax.lax.rem(jnp.int32(chunk), nb)).wait()

    return pl.pallas_call(
        body,
        grid=(n_chunks,),
        in_specs=[
            pl.BlockSpec(memory_space=pl.ANY),
            pl.BlockSpec(memory_space=pltpu.VMEM),
        ],
        out_specs=pl.BlockSpec(memory_space=pl.ANY),
        out_shape=jax.ShapeDtypeStruct((n_rows, tokens, _EMB), jnp.float32),
        scratch_shapes=[
            pltpu.VMEM((nb, cr, tokens, _EMB), jnp.float32),
            pltpu.VMEM((nb, cr, tokens, _EMB), jnp.float32),
            pltpu.SemaphoreType.DMA((nb,)),
            pltpu.SemaphoreType.DMA((nb,)),
        ],
        interpret=interpret,
    )


def kernel(x, blocks, emb_table):
    batch, num_blocks, tokens, emb = x.shape
    n_rows = batch * num_blocks
    idx = blocks.astype(jnp.int32).reshape(n_rows)
    enc = _sc_gather(n_rows)(emb_table, idx)
    out = _tc_add(n_rows, tokens)(
        x.reshape(n_rows, tokens, emb), enc.reshape(n_rows, 1, emb)
    )
    return out.reshape(batch, num_blocks * tokens, emb)
